# baseline (speedup 1.0000x reference)
import functools

import jax
import jax.numpy as jnp
from jax import lax
from jax.experimental import pallas as pl
from jax.experimental.pallas import tpu as pltpu
from jax.experimental.pallas import tpu_sc as plsc

NUM_GENRES = 20
EMBED_DIM = 240
TITLE_DIM = 768
MOVIE_FEAT_DIM = 64
BATCH = 16384

_NC = 2
_NS = 16
_NW = _NC * _NS
_B_PER_W = BATCH // _NW
_GROUPS = _B_PER_W // 16
_CHUNK_GROUPS = 8
_N_CHUNKS = _GROUPS // _CHUNK_GROUPS


_CHUNK_ROWS = _CHUNK_GROUPS * 16
_NCHUNK = EMBED_DIM // 16
_PAD_STRIDE = EMBED_DIM + 1


def _fast_rsqrt(s):
    i = plsc.bitcast(s, jnp.int32)
    i = jnp.int32(0x5F3759DF) - (i >> 1)
    y = plsc.bitcast(i, jnp.float32)
    for _ in range(3):
        y = y * (jnp.float32(1.5) - jnp.float32(0.5) * s * y * y)
    return y


def _sc_user_body(idx_hbm, tab_hbm, out_hbm,
                  idx_v, tab2d, tab_flat, inv_v, buf_a, buf_b, sem):
    wid = lax.axis_index("s") * _NC + lax.axis_index("c")
    base = wid * _B_PER_W
    lane = lax.iota(jnp.int32, 16)

    pltpu.sync_copy(idx_hbm.at[pl.ds(base, _B_PER_W)], idx_v)
    pltpu.sync_copy(tab_hbm, tab2d)

    def repack_body(r, _):
        for c in range(_NCHUNK):
            v = tab2d[r, pl.ds(c * 16, 16)]
            plsc.store_scatter(
                tab_flat, [r * _PAD_STRIDE + c * 16 + lane], v)
        return 0

    lax.fori_loop(0, NUM_GENRES, repack_body, 0)

    src0 = lane * _PAD_STRIDE
    src1 = jnp.minimum(lane + 16, NUM_GENRES - 1) * _PAD_STRIDE
    zero = jnp.zeros((16,), jnp.float32)

    def norm_body(c, carry):
        a0, a1 = carry
        v0 = plsc.load_gather(tab_flat, [src0 + c])
        v1 = plsc.load_gather(tab_flat, [src1 + c])
        return (a0 + v0 * v0, a1 + v1 * v1)

    acc0, acc1 = plsc.parallel_loop(
        0, EMBED_DIM, unroll=8, carry=(zero, zero))(norm_body)
    inv_v[pl.ds(0, 16)] = jnp.minimum(_fast_rsqrt(acc0), jnp.float32(1e12))
    inv_v[pl.ds(16, 16)] = jnp.minimum(_fast_rsqrt(acc1), jnp.float32(1e12))

    bufs = (buf_a, buf_b)
    copies = []
    for k in range(_N_CHUNKS):
        buf = bufs[k % 2]
        if k >= 2:
            copies[k - 2].wait()
        for j in range(_CHUNK_GROUPS):
            g = k * _CHUNK_GROUPS + j
            idxv = idx_v[pl.ds(g * 16, 16)]
            scale = plsc.load_gather(inv_v, [idxv])
            src = idxv * _PAD_STRIDE

            def col_body(d, j=j, src=src, scale=scale, buf=buf):
                v = plsc.load_gather(tab_flat, [src + d])
                buf[d, pl.ds(j * 16, 16)] = v * scale

            plsc.parallel_loop(0, EMBED_DIM, unroll=8)(col_body)
        copies.append(pltpu.async_copy(
            buf,
            out_hbm.at[:, pl.ds(base + k * _CHUNK_ROWS, _CHUNK_ROWS)],
            sem))
    for c in copies[-2:]:
        c.wait()


_sc_user_tower = functools.partial(
    pl.kernel,
    out_type=jax.ShapeDtypeStruct((EMBED_DIM, BATCH), jnp.float32),
    mesh=plsc.VectorSubcoreMesh(core_axis_name="c", subcore_axis_name="s"),
    scratch_types=[
        pltpu.VMEM((_B_PER_W,), jnp.int32),
        pltpu.VMEM((NUM_GENRES, EMBED_DIM), jnp.float32),
        pltpu.VMEM((NUM_GENRES * _PAD_STRIDE + 12,), jnp.float32),
        pltpu.VMEM((32,), jnp.float32),
        pltpu.VMEM((EMBED_DIM, _CHUNK_ROWS), jnp.float32),
        pltpu.VMEM((EMBED_DIM, _CHUNK_ROWS), jnp.float32),
        pltpu.SemaphoreType.DMA,
    ],
    compiler_params=pltpu.CompilerParams(needs_layout_passes=False),
)(_sc_user_body)


_BM = 4096


def _movie_body(title_ref, feat_t_ref, w_ref, b_ref, out_ref):
    w = w_ref[...]
    acc = lax.dot_general(
        w[:TITLE_DIM], title_ref[...],
        dimension_numbers=(((0,), (1,)), ((), ())),
        preferred_element_type=jnp.float32)
    acc = acc + lax.dot_general(
        w[TITLE_DIM:], feat_t_ref[...],
        dimension_numbers=(((0,), (0,)), ((), ())),
        preferred_element_type=jnp.float32)
    acc = acc + b_ref[...]
    norm = jnp.sqrt(jnp.sum(acc * acc, axis=0, keepdims=True))
    out_ref[...] = acc / jnp.maximum(norm, 1e-12)


def _movie_tower(title_embeddings, movie_features, W_movie, b_movie):
    feat_t = movie_features.T
    bias = b_movie.reshape(EMBED_DIM, 1)
    grid = (BATCH // _BM,)
    out_t = pl.pallas_call(
        _movie_body,
        grid=grid,
        in_specs=[
            pl.BlockSpec((_BM, TITLE_DIM), lambda i: (i, 0)),
            pl.BlockSpec((MOVIE_FEAT_DIM, _BM), lambda i: (0, i)),
            pl.BlockSpec((TITLE_DIM + MOVIE_FEAT_DIM, EMBED_DIM),
                         lambda i: (0, 0)),
            pl.BlockSpec((EMBED_DIM, 1), lambda i: (0, 0)),
        ],
        out_specs=pl.BlockSpec((EMBED_DIM, _BM), lambda i: (0, i)),
        out_shape=jax.ShapeDtypeStruct((EMBED_DIM, BATCH), jnp.float32),
    )(title_embeddings, feat_t, W_movie, bias)
    return out_t.T


def kernel(user_features, title_embeddings, movie_features, user_table, W_movie, b_movie):
    user_embedding = _sc_user_tower(user_features, user_table).T
    movie_embedding = _movie_tower(title_embeddings, movie_features, W_movie, b_movie)
    return (user_embedding, movie_embedding)
